# baseline (speedup 1.0000x reference)
import functools

import jax
import jax.numpy as jnp
from jax.experimental import pallas as pl

NUM_NODES = 100
GNN_OBS = 16
ACT = 2
HID = 128
K = 16
LOG_STD_MIN = -5.0
LOG_STD_MAX = 2.0

BB = 25


def _gcnn_kernel(feats_ref, w0s_ref, w0n_ref, b0_ref, w1s_ref, w1n_ref,
                 b1_ref, wm1_ref, bm1_ref, wm2_ref, bm2_ref, out_ref):
    f32 = jnp.float32
    feats = feats_ref[...]
    lx = feats[:, :, 0]
    ly = feats[:, :, 1]
    x = feats[:, :, ACT:].reshape(BB * NUM_NODES, GNN_OBS - ACT)

    dx = lx[:, :, None] - lx[:, None, :]
    dy = ly[:, :, None] - ly[:, None, :]
    d2 = dx * dx + dy * dy
    eye = (jax.lax.broadcasted_iota(jnp.int32, (NUM_NODES, NUM_NODES), 0)
           == jax.lax.broadcasted_iota(jnp.int32, (NUM_NODES, NUM_NODES), 1))
    d2 = d2 + jnp.where(eye, f32(1e9), f32(0.0))[None]

    kb = jax.lax.bitcast_convert_type(d2, jnp.int32)

    def body(it, T):
        cand = T | (jnp.int32(1) << (30 - it))
        cnt = jnp.sum((kb < cand).astype(jnp.int32), axis=1, keepdims=True)
        return jnp.where(cnt <= K - 1, cand, T)

    T = jax.lax.fori_loop(
        0, 31, body, jnp.zeros((BB, 1, NUM_NODES), jnp.int32))

    base = kb < T
    cnt_lt = jnp.sum(base.astype(jnp.int32), axis=1, keepdims=True)
    ties = kb == T
    ties_f = ties.astype(f32)
    ltri = (jax.lax.broadcasted_iota(jnp.int32, (BB, NUM_NODES, NUM_NODES), 1)
            > jax.lax.broadcasted_iota(jnp.int32, (BB, NUM_NODES, NUM_NODES), 2)
            ).astype(f32)
    cum = jax.lax.dot_general(
        ltri, ties_f, (((2,), (1,)), ((0,), (0,))),
        preferred_element_type=f32)
    adj_t = (base | (ties & (cum < (K - cnt_lt).astype(f32)))).astype(f32)

    bf16 = jnp.bfloat16

    def dot(a, b):
        return jnp.dot(a.astype(bf16), b.astype(bf16),
                       preferred_element_type=f32)

    def layer(xin, wself, wnei, bias):
        h = dot(xin, wnei)
        hb = h.reshape(BB, NUM_NODES, HID)
        agg = jax.lax.dot_general(
            adj_t.astype(bf16), hb.astype(bf16),
            (((1,), (1,)), ((0,), (0,))),
            preferred_element_type=f32)
        agg = agg.reshape(BB * NUM_NODES, HID)
        return jax.nn.relu(dot(xin, wself) + agg + bias[None, :])

    x = layer(x, w0s_ref[...], w0n_ref[...], b0_ref[...])
    x = layer(x, w1s_ref[...], w1n_ref[...], b1_ref[...])
    h = jax.nn.relu(dot(x, wm1_ref[...]) + bm1_ref[...][None, :])
    out = dot(h, wm2_ref[...]) + bm2_ref[...][None, :]
    mu = out[:, :ACT]
    log_std = jnp.tanh(out[:, ACT:])
    log_std = LOG_STD_MIN + 0.5 * (LOG_STD_MAX - LOG_STD_MIN) * (log_std + 1.0)
    res = jnp.concatenate([jnp.tanh(mu), jnp.exp(log_std)], axis=-1)
    out_ref[...] = res.reshape(BB, NUM_NODES, 2 * ACT)


def kernel(obs, W0_self, W0_nei, b0, W1_self, W1_nei, b1, Wm1, bm1, Wm2, bm2):
    bs = obs.shape[0]
    feats = obs.reshape(bs, NUM_NODES, GNN_OBS)
    grid = (bs // BB,)
    wspec = lambda *shape: pl.BlockSpec(shape, lambda i: (0,) * len(shape))
    out = pl.pallas_call(
        _gcnn_kernel,
        grid=grid,
        in_specs=[
            pl.BlockSpec((BB, NUM_NODES, GNN_OBS), lambda i: (i, 0, 0)),
            wspec(GNN_OBS - ACT, HID), wspec(GNN_OBS - ACT, HID), wspec(HID),
            wspec(HID, HID), wspec(HID, HID), wspec(HID),
            wspec(HID, HID), wspec(HID),
            wspec(HID, 2 * ACT), wspec(2 * ACT),
        ],
        out_specs=pl.BlockSpec((BB, NUM_NODES, 2 * ACT), lambda i: (i, 0, 0)),
        out_shape=jax.ShapeDtypeStruct((bs, NUM_NODES, 2 * ACT), jnp.float32),
    )(feats, W0_self, W0_nei, b0, W1_self, W1_nei, b1, Wm1, bm1, Wm2, bm2)
    return out.reshape(bs * NUM_NODES, 2 * ACT)


# speedup vs baseline: 1.0418x; 1.0418x over previous
import functools

import jax
import jax.numpy as jnp
from jax.experimental import pallas as pl

NUM_NODES = 100
GNN_OBS = 16
ACT = 2
HID = 128
K = 16
LOG_STD_MIN = -5.0
LOG_STD_MAX = 2.0

BB = 50


def _gcnn_kernel(feats_ref, w0s_ref, w0n_ref, b0_ref, w1s_ref, w1n_ref,
                 b1_ref, wm1_ref, bm1_ref, wm2_ref, bm2_ref, out_ref):
    f32 = jnp.float32
    feats = feats_ref[...]
    lx = feats[:, :, 0]
    ly = feats[:, :, 1]
    x = feats[:, :, ACT:].reshape(BB * NUM_NODES, GNN_OBS - ACT)

    dx = lx[:, :, None] - lx[:, None, :]
    dy = ly[:, :, None] - ly[:, None, :]
    d2 = dx * dx + dy * dy
    eye = (jax.lax.broadcasted_iota(jnp.int32, (NUM_NODES, NUM_NODES), 0)
           == jax.lax.broadcasted_iota(jnp.int32, (NUM_NODES, NUM_NODES), 1))
    d2 = d2 + jnp.where(eye, f32(1e9), f32(0.0))[None]

    kb = jax.lax.bitcast_convert_type(d2, jnp.int32)

    def body(it, T):
        cand = T | (jnp.int32(1) << (30 - it))
        cnt = jnp.sum((kb < cand).astype(jnp.int32), axis=1, keepdims=True)
        return jnp.where(cnt <= K - 1, cand, T)

    T = jax.lax.fori_loop(
        0, 31, body, jnp.zeros((BB, 1, NUM_NODES), jnp.int32))

    base = kb < T
    cnt_lt = jnp.sum(base.astype(jnp.int32), axis=1, keepdims=True)
    ties = kb == T
    ties_f = ties.astype(f32)
    ltri = (jax.lax.broadcasted_iota(jnp.int32, (BB, NUM_NODES, NUM_NODES), 1)
            > jax.lax.broadcasted_iota(jnp.int32, (BB, NUM_NODES, NUM_NODES), 2)
            ).astype(f32)
    cum = jax.lax.dot_general(
        ltri, ties_f, (((2,), (1,)), ((0,), (0,))),
        preferred_element_type=f32)
    adj_t = (base | (ties & (cum < (K - cnt_lt).astype(f32)))).astype(f32)

    dot = functools.partial(jnp.dot, preferred_element_type=f32)

    def layer(xin, wself, wnei, bias):
        h = dot(xin, wnei)
        hb = h.reshape(BB, NUM_NODES, HID)
        agg = jax.lax.dot_general(
            adj_t, hb, (((1,), (1,)), ((0,), (0,))),
            preferred_element_type=f32)
        agg = agg.reshape(BB * NUM_NODES, HID)
        return jax.nn.relu(dot(xin, wself) + agg + bias[None, :])

    x = layer(x, w0s_ref[...], w0n_ref[...], b0_ref[...])
    x = layer(x, w1s_ref[...], w1n_ref[...], b1_ref[...])
    h = jax.nn.relu(dot(x, wm1_ref[...]) + bm1_ref[...][None, :])
    out = dot(h, wm2_ref[...]) + bm2_ref[...][None, :]
    mu = out[:, :ACT]
    log_std = jnp.tanh(out[:, ACT:])
    log_std = LOG_STD_MIN + 0.5 * (LOG_STD_MAX - LOG_STD_MIN) * (log_std + 1.0)
    res = jnp.concatenate([jnp.tanh(mu), jnp.exp(log_std)], axis=-1)
    out_ref[...] = res.reshape(BB, NUM_NODES, 2 * ACT)


def kernel(obs, W0_self, W0_nei, b0, W1_self, W1_nei, b1, Wm1, bm1, Wm2, bm2):
    bs = obs.shape[0]
    feats = obs.reshape(bs, NUM_NODES, GNN_OBS)
    grid = (bs // BB,)
    wspec = lambda *shape: pl.BlockSpec(shape, lambda i: (0,) * len(shape))
    out = pl.pallas_call(
        _gcnn_kernel,
        grid=grid,
        in_specs=[
            pl.BlockSpec((BB, NUM_NODES, GNN_OBS), lambda i: (i, 0, 0)),
            wspec(GNN_OBS - ACT, HID), wspec(GNN_OBS - ACT, HID), wspec(HID),
            wspec(HID, HID), wspec(HID, HID), wspec(HID),
            wspec(HID, HID), wspec(HID),
            wspec(HID, 2 * ACT), wspec(2 * ACT),
        ],
        out_specs=pl.BlockSpec((BB, NUM_NODES, 2 * ACT), lambda i: (i, 0, 0)),
        out_shape=jax.ShapeDtypeStruct((bs, NUM_NODES, 2 * ACT), jnp.float32),
    )(feats, W0_self, W0_nei, b0, W1_self, W1_nei, b1, Wm1, bm1, Wm2, bm2)
    return out.reshape(bs * NUM_NODES, 2 * ACT)
